# baseline (speedup 1.0000x reference)
import functools

import jax
import jax.numpy as jnp
from jax import lax
from jax.experimental import pallas as pl
from jax.experimental.pallas import tpu as pltpu
from jax.experimental.pallas import tpu_sc as plsc

N_NODES = 100000
D = 128
M = 200000
K = 3

_LANES = 16
_B = 64
_NSETS = 4


def _build(nc: int, ns: int):
    nw = nc * ns
    mpw = M // nw
    n_full = mpw // _B
    tail = mpw - n_full * _B
    n_chunk = n_full + (1 if tail else 0)
    assert tail and n_full >= 2 * _NSETS

    mesh = plsc.VectorSubcoreMesh(core_axis_name="c", subcore_axis_name="s")

    buf = lambda: pltpu.VMEM((_B, D), jnp.float32)

    @functools.partial(
        pl.kernel,
        mesh=mesh,
        compiler_params=pltpu.CompilerParams(use_tc_tiling_on_sc=False),
        out_type=jax.ShapeDtypeStruct((M, D), jnp.float32),
        scratch_types=(
            [pltpu.VMEM((K, n_chunk, _B), jnp.int32)]
            + [buf() for _ in range(3 * _NSETS)]
            + [pltpu.SemaphoreType.DMA for _ in range(2 * _NSETS)]
        ),
    )
    def sc_kernel(x_hbm, idx_hbm, out_hbm, idx_v, *bufs_and_sems):
        wid = lax.axis_index("s") * nc + lax.axis_index("c")
        base = wid * mpw
        pltpu.sync_copy(idx_hbm.at[wid], idx_v)

        bufs = bufs_and_sems[: 3 * _NSETS]
        sems = bufs_and_sems[3 * _NSETS:]
        sets = [
            (bufs[3 * p], bufs[3 * p + 1], bufs[3 * p + 2],
             sems[2 * p], sems[2 * p + 1])
            for p in range(_NSETS)
        ]

        def fire_gathers(c, st):
            ob, b1, b2, semg, _ = st
            pltpu.async_copy(x_hbm.at[idx_v.at[0, c]], ob, semg)
            pltpu.async_copy(x_hbm.at[idx_v.at[1, c]], b1, semg)
            pltpu.async_copy(x_hbm.at[idx_v.at[2, c]], b2, semg)

        def wait_gathers(c, st):
            ob, b1, b2, semg, _ = st
            pltpu.make_async_copy(x_hbm.at[idx_v.at[0, c]], ob, semg).wait()
            pltpu.make_async_copy(x_hbm.at[idx_v.at[1, c]], b1, semg).wait()
            pltpu.make_async_copy(x_hbm.at[idx_v.at[2, c]], b2, semg).wait()

        def fire_store(c, st):
            pltpu.async_copy(st[0], out_hbm.at[pl.ds(base + c * _B, _B)], st[4])

        def wait_store(c, st):
            pltpu.make_async_copy(
                st[0], out_hbm.at[pl.ds(base + c * _B, _B)], st[4]).wait()

        def accumulate(st):
            ob, b1, b2 = st[0], st[1], st[2]

            def row(r, carry):
                for j in range(D // _LANES):
                    sl = pl.ds(j * _LANES, _LANES)
                    plsc.addupdate(ob.at[r, sl], b1[r, sl] + b2[r, sl])
                return carry

            lax.fori_loop(0, _B, row, 0)

        def step(c, phase, fire=True, wait_prev=True):
            prev = sets[(phase + _NSETS - 1) % _NSETS]
            if wait_prev:
                wait_store(c - 1, prev)
            if fire:
                fire_gathers(c + (_NSETS - 1), prev)
            cur = sets[phase]
            wait_gathers(c, cur)
            accumulate(cur)
            fire_store(c, cur)

        for c0 in range(_NSETS - 1):
            fire_gathers(c0, sets[c0])
        step(0, 0, wait_prev=False)

        npair = (n_full - 1 - 2) // _NSETS
        def body(j, carry):
            for i in range(_NSETS):
                c = _NSETS * j + 1 + i
                step(c, (1 + i) % _NSETS)
            return carry
        lax.fori_loop(0, npair, body, 0)

        c = _NSETS * npair + 1
        while c < n_full:
            step(c, c % _NSETS, fire=(c + _NSETS - 1 <= n_full))
            c += 1

        st = sets[n_full % _NSETS]
        wait_store(n_full - 1, sets[(n_full - 1) % _NSETS])
        wait_gathers(n_full, st)
        accumulate(st)
        pltpu.sync_copy(
            st[0].at[pl.ds(0, tail)],
            out_hbm.at[pl.ds(base + n_full * _B, tail)],
        )

    def run(x, indices):
        idx32 = indices.astype(jnp.int32)
        idx_t = idx32.T.reshape(K, nw, mpw).transpose(1, 0, 2)
        pad = n_chunk * _B - mpw
        if pad:
            idx_t = jnp.pad(idx_t, ((0, 0), (0, 0), (0, pad)))
        idx_t = idx_t.reshape(nw, K, n_chunk, _B)
        return sc_kernel(x, idx_t)

    return run


def kernel(x, indices):
    info = plsc.get_sparse_core_info()
    return _build(info.num_cores, info.num_subcores)(x, indices)


# speedup vs baseline: 1.5442x; 1.5442x over previous
import functools

import jax
import jax.numpy as jnp
from jax import lax
from jax.experimental import pallas as pl
from jax.experimental.pallas import tpu as pltpu
from jax.experimental.pallas import tpu_sc as plsc

N_NODES = 100000
D = 128
M = 200000
K = 3

_LANES = 16
_B = 128


def _build(nc: int, ns: int):
    nw = nc * ns
    w_lo = ((M // nw) // 8) * 8
    extra = M - nw * w_lo
    assert extra % 8 == 0 and extra // 8 <= nw
    n_hi = extra // 8
    t0 = nw - n_hi
    w_hi = w_lo + 8
    n_full = w_lo // _B
    hi_tail = w_hi - n_full * _B
    tb = -(-hi_tail // _LANES) * _LANES
    assert 0 < tb <= _B and tb <= n_full * _B and tb % 8 == 0
    assert n_full >= 8

    mesh = plsc.VectorSubcoreMesh(core_axis_name="c", subcore_axis_name="s")

    @functools.partial(
        pl.kernel,
        mesh=mesh,
        compiler_params=pltpu.CompilerParams(use_tc_tiling_on_sc=False),
        out_type=jax.ShapeDtypeStruct((M, D), jnp.float32),
        scratch_types=(
            [pltpu.VMEM((_B, D), jnp.float32) for _ in range(6)]
            + [pltpu.VMEM((K, _B), jnp.int32) for _ in range(4)]
            + [pltpu.SemaphoreType.DMA for _ in range(8)]
        ),
    )
    def sc_kernel(x_hbm, idx_hbm, out_hbm, *refs):
        wid = lax.axis_index("s") * nc + lax.axis_index("c")
        base = wid * w_lo + 8 * jnp.maximum(wid - t0, 0)
        mpw = w_lo + 8 * (wid >= t0).astype(jnp.int32)

        bufs, cols, sems = refs[:6], refs[6:10], refs[10:]
        dsets = [(bufs[3 * p], bufs[3 * p + 1], bufs[3 * p + 2],
                  sems[2 * p], sems[2 * p + 1]) for p in range(2)]
        isets = [(cols[p], sems[4 + p]) for p in range(4)]
        S = lambda j: dsets[j % 2]
        I = lambda j: isets[j % 4]

        def row0(c):
            if isinstance(c, int) and c == n_full:
                return mpw - tb
            return c * _B

        def idx_copies(c, iset, n):
            col, semi = iset
            r0 = base + row0(c)
            return [
                pltpu.make_async_copy(
                    idx_hbm.at[k, pl.ds(r0, n)],
                    col.at[k] if n == _B else col.at[k, pl.ds(0, n)],
                    semi)
                for k in range(K)
            ]

        def gather_copies(st, iset, n):
            col = iset[0]
            return [
                pltpu.make_async_copy(
                    x_hbm.at[col.at[k] if n == _B else col.at[k, pl.ds(0, n)]],
                    dst if n == _B else dst.at[pl.ds(0, n)],
                    st[3])
                for k, dst in enumerate((st[0], st[1], st[2]))
            ]

        def store_copy(c, st, n):
            src = st[0] if n == _B else st[0].at[pl.ds(0, n)]
            return pltpu.make_async_copy(
                src, out_hbm.at[pl.ds(base + row0(c), n)], st[4])

        def fire(descs):
            for d in descs:
                d.start()

        def drain(descs):
            for d in descs:
                d.wait()

        def accumulate(st, n=_B):
            ob, b1, b2 = st[0], st[1], st[2]

            def rows(r2, carry):
                for u in range(2):
                    r = r2 * 2 + u
                    for j in range(D // _LANES):
                        sl = pl.ds(j * _LANES, _LANES)
                        plsc.addupdate(ob.at[r, sl], b1[r, sl] + b2[r, sl])
                return carry

            lax.fori_loop(0, n // 2, rows, 0)

        def steady(c, cm, n2=_B):
            cur, oth = S(cm), S(cm + 1)
            drain(idx_copies(c + 1, I(cm + 1), _B))
            drain([store_copy(c - 1, oth, _B)])
            fire(gather_copies(oth, I(cm + 1), _B))
            fire(idx_copies(c + 2 if n2 == _B else n_full, I(cm + 2), n2))
            drain(gather_copies(cur, I(cm), _B))
            accumulate(cur)
            fire([store_copy(c, cur, _B)])

        fire(idx_copies(0, I(0), _B))
        fire(idx_copies(1, I(1), _B))
        drain(idx_copies(0, I(0), _B))
        fire(gather_copies(S(0), I(0), _B))
        fire(idx_copies(2, I(2), _B))
        drain(idx_copies(1, I(1), _B))
        fire(gather_copies(S(1), I(1), _B))
        fire(idx_copies(3, I(3), _B))
        drain(gather_copies(S(0), I(0), _B))
        accumulate(S(0))
        fire([store_copy(0, S(0), _B)])

        n_steady = n_full - 2
        def body(m, carry):
            for i in range(4):
                steady(4 * m + 1 + i, 1 + i)
            return carry
        lax.fori_loop(0, (n_steady - 1) // 4, body, 0)
        for j in range(4 * ((n_steady - 1) // 4) + 1, n_steady + 1):
            steady(j, j, n2=(_B if j + 2 <= n_full - 1 else tb))

        c = n_full - 1
        drain(idx_copies(n_full, I(n_full), tb))
        drain([store_copy(c - 1, S(c + 1), _B)])
        fire(gather_copies(S(c + 1), I(n_full), tb))
        drain(gather_copies(S(c), I(c), _B))
        accumulate(S(c))
        fire([store_copy(c, S(c), _B)])

        st = S(n_full)
        drain(gather_copies(st, I(n_full), tb))
        accumulate(st, n=tb)
        drain([store_copy(n_full - 1, S(n_full - 1), _B)])
        pltpu.sync_copy(st[0].at[pl.ds(0, tb)],
                        out_hbm.at[pl.ds(base + row0(n_full), tb)])

    def run(x, indices):
        return sc_kernel(x, indices.astype(jnp.int32).T)

    return run


def kernel(x, indices):
    info = plsc.get_sparse_core_info()
    return _build(info.num_cores, info.num_subcores)(x, indices)
